# baseline (speedup 1.0000x reference)
import functools

import jax
import jax.numpy as jnp
from jax.experimental import pallas as pl
from jax.experimental.pallas import tpu as pltpu

_BF = jnp.bfloat16
_BB = 2


def _im2col_s2(h, win_ref, bb, hw):
    hv = h.reshape(bb, hw, 2, hw, 256)
    for a in (0, 1):
        for b in (0, 1):
            for dh in (0, 1):
                for dw in (0, 1):
                    c0 = 512 * (2 * a + b) + 128 * (2 * dh + dw)
                    ro, co = a + dh - 1, b + dw - 1
                    rlo, rhi = max(0, -ro), hw - max(0, ro)
                    clo, chi = max(0, -co), hw - max(0, co)
                    if rlo > 0:
                        win_ref[:, 0:rlo, :, c0:c0 + 128] = jnp.zeros(
                            (bb, rlo, hw, 128), _BF)
                    if rhi < hw:
                        win_ref[:, rhi:hw, :, c0:c0 + 128] = jnp.zeros(
                            (bb, hw - rhi, hw, 128), _BF)
                    if clo > 0:
                        win_ref[:, :, 0:clo, c0:c0 + 128] = jnp.zeros(
                            (bb, hw, clo, 128), _BF)
                    if chi < hw:
                        win_ref[:, :, chi:hw, c0:c0 + 128] = jnp.zeros(
                            (bb, hw, hw - chi, 128), _BF)
                    win_ref[:, rlo:rhi, clo:chi, c0:c0 + 128] = hv[
                        :, rlo + ro:rhi + ro, 1 - dh,
                        clo + co:chi + co, (1 - dw) * 128:(2 - dw) * 128]


def _im2col_3x3(hr, rwin_ref, bb):
    for kh in range(3):
        for kw in range(3):
            c0 = 128 * (3 * kh + kw)
            ro, co = kh - 1, kw - 1
            rlo, rhi = max(0, -ro), 16 - max(0, ro)
            clo, chi = max(0, -co), 16 - max(0, co)
            if rlo > 0:
                rwin_ref[:, 0:rlo, :, c0:c0 + 128] = jnp.zeros(
                    (bb, rlo, 16, 128), _BF)
            if rhi < 16:
                rwin_ref[:, rhi:16, :, c0:c0 + 128] = jnp.zeros(
                    (bb, 16 - rhi, 16, 128), _BF)
            if clo > 0:
                rwin_ref[:, :, 0:clo, c0:c0 + 128] = jnp.zeros(
                    (bb, 16, clo, 128), _BF)
            if chi < 16:
                rwin_ref[:, :, chi:16, c0:c0 + 128] = jnp.zeros(
                    (bb, 16, 16 - chi, 128), _BF)
            rwin_ref[:, rlo:rhi, clo:chi, c0:c0 + 128] = hr[
                :, rlo + ro:rhi + ro, clo + co:chi + co, :]


def _mega_body(p1_ref, w1_ref, b1_ref, w2_ref, b2_ref, w3_ref, b3_ref,
               r0w1_ref, r0w2_ref, r1w1_ref, r1w2_ref, o_ref,
               h1_ref, win2_ref, win3_ref, rwin_ref, *, bb):
    acc = jnp.dot(p1_ref[...].reshape(bb * 4096, 48), w1_ref[...],
                  preferred_element_type=jnp.float32)
    acc = jnp.maximum(acc + b1_ref[...], 0.0)
    h1_ref[...] = acc.reshape(bb, 64, 64, 128).astype(_BF)

    _im2col_s2(h1_ref[...], win2_ref, bb, 32)
    acc = jnp.dot(win2_ref[...].reshape(bb * 1024, 2048), w2_ref[...],
                  preferred_element_type=jnp.float32)
    h2 = jnp.maximum(acc + b2_ref[...], 0.0).astype(_BF).reshape(bb, 32, 32, 128)

    _im2col_s2(h2, win3_ref, bb, 16)
    acc = jnp.dot(win3_ref[...].reshape(bb * 256, 2048), w3_ref[...],
                  preferred_element_type=jnp.float32)
    h = acc + b3_ref[...]

    for w1_ref, w2_ref, relu_out in ((r0w1_ref, r0w2_ref, False),
                                     (r1w1_ref, r1w2_ref, True)):
        hr = jnp.maximum(h, 0.0).astype(_BF).reshape(bb, 16, 16, 128)
        _im2col_3x3(hr, rwin_ref, bb)
        t = jnp.dot(rwin_ref[...].reshape(bb * 256, 1152), w1_ref[...],
                    preferred_element_type=jnp.float32)
        t = jnp.maximum(t, 0.0).astype(_BF)
        h = h + jnp.dot(t, w2_ref[...], preferred_element_type=jnp.float32)
        if relu_out:
            h = jnp.maximum(h, 0.0)

    o_ref[...] = jnp.transpose(h.reshape(bb, 256, 128), (0, 2, 1))


def kernel(x_nchw, c1_w, c1_b, c2_w, c2_b, c3_w, c3_b,
           res0_w1, res0_w2, res1_w1, res1_w2):
    B = x_nchw.shape[0]
    xp = jnp.pad(x_nchw.astype(_BF), ((0, 0), (0, 0), (1, 1), (1, 1)))
    taps = [jax.lax.slice(xp, (0, 0, kh, kw), (B, 3, kh + 127, kw + 127),
                          (1, 1, 2, 2))
            for kh in range(4) for kw in range(4)]
    pt = jnp.stack(taps, axis=1)
    p1 = jnp.transpose(pt, (0, 3, 4, 1, 2)).reshape(B, 4096, 48)

    w1 = (c1_w.reshape(2, 2, 2, 2, 3, 128).transpose(0, 2, 1, 3, 4, 5)
          .reshape(48, 128).astype(_BF))
    w2 = c2_w.astype(_BF)
    w3 = c3_w.astype(_BF)
    r0w1 = res0_w1.astype(_BF)
    r0w2 = res0_w2.astype(_BF)
    r1w1 = res1_w1.astype(_BF)
    r1w2 = res1_w2.astype(_BF)

    full = lambda shp: pl.BlockSpec(shp, lambda i: (0,) * len(shp))

    out = pl.pallas_call(
        functools.partial(_mega_body, bb=_BB),
        grid=(B // _BB,),
        in_specs=[
            pl.BlockSpec((_BB, 4096, 48), lambda i: (i, 0, 0)),
            full((48, 128)), full((1, 128)),
            full((2048, 128)), full((1, 128)),
            full((2048, 128)), full((1, 128)),
            full((1152, 128)), full((128, 128)),
            full((1152, 128)), full((128, 128)),
        ],
        out_shape=jax.ShapeDtypeStruct((B, 128, 256), jnp.float32),
        out_specs=pl.BlockSpec((_BB, 128, 256), lambda i: (i, 0, 0)),
        scratch_shapes=[
            pltpu.VMEM((_BB, 64, 64, 128), _BF),
            pltpu.VMEM((_BB, 32, 32, 2048), _BF),
            pltpu.VMEM((_BB, 16, 16, 2048), _BF),
            pltpu.VMEM((_BB, 16, 16, 1152), _BF),
        ],
        compiler_params=pltpu.CompilerParams(
            dimension_semantics=("parallel",)),
    )(p1, w1, c1_b, w2, c2_b, w3, c3_b, r0w1, r0w2, r1w1, r1w2)

    return out.reshape(B, 128, 16, 16)


# speedup vs baseline: 2.5107x; 2.5107x over previous
import functools

import jax
import jax.numpy as jnp
from jax.experimental import pallas as pl
from jax.experimental.pallas import tpu as pltpu

_BF = jnp.bfloat16
_BB = 2


def _im2col_s2(h, win_ref, bb, hw):
    hv = h.reshape(bb, hw, 2, hw, 256)
    for a in (0, 1):
        for b in (0, 1):
            for dh in (0, 1):
                for dw in (0, 1):
                    c0 = 512 * (2 * a + b) + 128 * (2 * dh + dw)
                    ro, co = a + dh - 1, b + dw - 1
                    rlo, rhi = max(0, -ro), hw - max(0, ro)
                    clo, chi = max(0, -co), hw - max(0, co)
                    if rlo > 0:
                        win_ref[:, 0:rlo, :, c0:c0 + 128] = jnp.zeros(
                            (bb, rlo, hw, 128), _BF)
                    if rhi < hw:
                        win_ref[:, rhi:hw, :, c0:c0 + 128] = jnp.zeros(
                            (bb, hw - rhi, hw, 128), _BF)
                    if clo > 0:
                        win_ref[:, :, 0:clo, c0:c0 + 128] = jnp.zeros(
                            (bb, hw, clo, 128), _BF)
                    if chi < hw:
                        win_ref[:, :, chi:hw, c0:c0 + 128] = jnp.zeros(
                            (bb, hw, hw - chi, 128), _BF)
                    win_ref[:, rlo:rhi, clo:chi, c0:c0 + 128] = hv[
                        :, rlo + ro:rhi + ro, 1 - dh,
                        clo + co:chi + co, (1 - dw) * 128:(2 - dw) * 128]


def _im2col_3x3(hr, rwin_ref, bb):
    for kh in range(3):
        for kw in range(3):
            c0 = 128 * (3 * kh + kw)
            ro, co = kh - 1, kw - 1
            rlo, rhi = max(0, -ro), 16 - max(0, ro)
            clo, chi = max(0, -co), 16 - max(0, co)
            if rlo > 0:
                rwin_ref[:, 0:rlo, :, c0:c0 + 128] = jnp.zeros(
                    (bb, rlo, 16, 128), _BF)
            if rhi < 16:
                rwin_ref[:, rhi:16, :, c0:c0 + 128] = jnp.zeros(
                    (bb, 16 - rhi, 16, 128), _BF)
            if clo > 0:
                rwin_ref[:, :, 0:clo, c0:c0 + 128] = jnp.zeros(
                    (bb, 16, clo, 128), _BF)
            if chi < 16:
                rwin_ref[:, :, chi:16, c0:c0 + 128] = jnp.zeros(
                    (bb, 16, 16 - chi, 128), _BF)
            rwin_ref[:, rlo:rhi, clo:chi, c0:c0 + 128] = hr[
                :, rlo + ro:rhi + ro, clo + co:chi + co, :]


def _mega_body(p1_ref, w1_ref, b1_ref, w2_ref, b2_ref, w3_ref, b3_ref,
               r0w1_ref, r0w2_ref, r1w1_ref, r1w2_ref, o_ref,
               h1_ref, win2_ref, win3_ref, rwin_ref, *, bb):
    acc = jnp.dot(p1_ref[...].reshape(bb * 4096, 48), w1_ref[...],
                  preferred_element_type=jnp.float32)
    acc = jnp.maximum(acc + b1_ref[...], 0.0)
    h1_ref[...] = acc.reshape(bb, 64, 64, 128).astype(_BF)

    _im2col_s2(h1_ref[...], win2_ref, bb, 32)
    acc = jnp.dot(win2_ref[...].reshape(bb * 1024, 2048), w2_ref[...],
                  preferred_element_type=jnp.float32)
    h2 = jnp.maximum(acc + b2_ref[...], 0.0).astype(_BF).reshape(bb, 32, 32, 128)

    _im2col_s2(h2, win3_ref, bb, 16)
    acc = jnp.dot(win3_ref[...].reshape(bb * 256, 2048), w3_ref[...],
                  preferred_element_type=jnp.float32)
    h = acc + b3_ref[...]

    for w1_ref, w2_ref, relu_out in ((r0w1_ref, r0w2_ref, False),
                                     (r1w1_ref, r1w2_ref, True)):
        hr = jnp.maximum(h, 0.0).astype(_BF).reshape(bb, 16, 16, 128)
        _im2col_3x3(hr, rwin_ref, bb)
        t = jnp.dot(rwin_ref[...].reshape(bb * 256, 1152), w1_ref[...],
                    preferred_element_type=jnp.float32)
        t = jnp.maximum(t, 0.0).astype(_BF)
        h = h + jnp.dot(t, w2_ref[...], preferred_element_type=jnp.float32)
        if relu_out:
            h = jnp.maximum(h, 0.0)

    o_ref[...] = jnp.transpose(h.reshape(bb, 256, 128), (0, 2, 1))


def kernel(x_nchw, c1_w, c1_b, c2_w, c2_b, c3_w, c3_b,
           res0_w1, res0_w2, res1_w1, res1_w2):
    B = x_nchw.shape[0]
    h = jnp.transpose(x_nchw, (0, 2, 3, 1))
    hp = jnp.pad(h, ((0, 0), (1, 1), (1, 1), (0, 0)))
    hp = hp.reshape(B, 65, 2, 65, 2, 3)
    xs1 = jnp.transpose(hp, (0, 1, 3, 2, 4, 5)).reshape(B, 65, 65, 12)
    xs1 = xs1.astype(_BF)
    cols = [xs1[:, a:a + 64, b:b + 64, :] for a in range(2) for b in range(2)]
    p1 = jnp.concatenate(cols, axis=-1).reshape(B, 4096, 48)

    w1 = c1_w.astype(_BF)
    w2 = c2_w.astype(_BF)
    w3 = c3_w.astype(_BF)
    r0w1 = res0_w1.astype(_BF)
    r0w2 = res0_w2.astype(_BF)
    r1w1 = res1_w1.astype(_BF)
    r1w2 = res1_w2.astype(_BF)

    full = lambda shp: pl.BlockSpec(shp, lambda i: (0,) * len(shp))

    out = pl.pallas_call(
        functools.partial(_mega_body, bb=_BB),
        grid=(B // _BB,),
        in_specs=[
            pl.BlockSpec((_BB, 4096, 48), lambda i: (i, 0, 0)),
            full((48, 128)), full((1, 128)),
            full((2048, 128)), full((1, 128)),
            full((2048, 128)), full((1, 128)),
            full((1152, 128)), full((128, 128)),
            full((1152, 128)), full((128, 128)),
        ],
        out_shape=jax.ShapeDtypeStruct((B, 128, 256), jnp.float32),
        out_specs=pl.BlockSpec((_BB, 128, 256), lambda i: (i, 0, 0)),
        scratch_shapes=[
            pltpu.VMEM((_BB, 64, 64, 128), _BF),
            pltpu.VMEM((_BB, 32, 32, 2048), _BF),
            pltpu.VMEM((_BB, 16, 16, 2048), _BF),
            pltpu.VMEM((_BB, 16, 16, 1152), _BF),
        ],
        compiler_params=pltpu.CompilerParams(
            dimension_semantics=("parallel",)),
    )(p1, w1, c1_b, w2, c2_b, w3, c3_b, r0w1, r0w2, r1w1, r1w2)

    return out.reshape(B, 128, 16, 16)


# speedup vs baseline: 2.5684x; 1.0230x over previous
import functools

import jax
import jax.numpy as jnp
from jax.experimental import pallas as pl
from jax.experimental.pallas import tpu as pltpu

_BF = jnp.bfloat16
_BB = 4


def _im2col_s2(h, win_ref, bb, hw):
    hv = h.reshape(bb, hw, 2, hw, 256)
    for a in (0, 1):
        for b in (0, 1):
            for dh in (0, 1):
                for dw in (0, 1):
                    c0 = 512 * (2 * a + b) + 128 * (2 * dh + dw)
                    ro, co = a + dh - 1, b + dw - 1
                    rlo, rhi = max(0, -ro), hw - max(0, ro)
                    clo, chi = max(0, -co), hw - max(0, co)
                    if rlo > 0:
                        win_ref[:, 0:rlo, :, c0:c0 + 128] = jnp.zeros(
                            (bb, rlo, hw, 128), _BF)
                    if rhi < hw:
                        win_ref[:, rhi:hw, :, c0:c0 + 128] = jnp.zeros(
                            (bb, hw - rhi, hw, 128), _BF)
                    if clo > 0:
                        win_ref[:, :, 0:clo, c0:c0 + 128] = jnp.zeros(
                            (bb, hw, clo, 128), _BF)
                    if chi < hw:
                        win_ref[:, :, chi:hw, c0:c0 + 128] = jnp.zeros(
                            (bb, hw, hw - chi, 128), _BF)
                    win_ref[:, rlo:rhi, clo:chi, c0:c0 + 128] = hv[
                        :, rlo + ro:rhi + ro, 1 - dh,
                        clo + co:chi + co, (1 - dw) * 128:(2 - dw) * 128]


def _im2col_3x3(hr, rwin_ref, bb):
    for kh in range(3):
        for kw in range(3):
            c0 = 128 * (3 * kh + kw)
            ro, co = kh - 1, kw - 1
            rlo, rhi = max(0, -ro), 16 - max(0, ro)
            clo, chi = max(0, -co), 16 - max(0, co)
            if rlo > 0:
                rwin_ref[:, 0:rlo, :, c0:c0 + 128] = jnp.zeros(
                    (bb, rlo, 16, 128), _BF)
            if rhi < 16:
                rwin_ref[:, rhi:16, :, c0:c0 + 128] = jnp.zeros(
                    (bb, 16 - rhi, 16, 128), _BF)
            if clo > 0:
                rwin_ref[:, :, 0:clo, c0:c0 + 128] = jnp.zeros(
                    (bb, 16, clo, 128), _BF)
            if chi < 16:
                rwin_ref[:, :, chi:16, c0:c0 + 128] = jnp.zeros(
                    (bb, 16, 16 - chi, 128), _BF)
            rwin_ref[:, rlo:rhi, clo:chi, c0:c0 + 128] = hr[
                :, rlo + ro:rhi + ro, clo + co:chi + co, :]


def _mega_body(p1_ref, w1_ref, b1_ref, w2_ref, b2_ref, w3_ref, b3_ref,
               r0w1_ref, r0w2_ref, r1w1_ref, r1w2_ref, o_ref,
               h1_ref, win2_ref, win3_ref, rwin_ref, *, bb):
    acc = jnp.dot(p1_ref[...].reshape(bb * 4096, 48), w1_ref[...],
                  preferred_element_type=jnp.float32)
    acc = jnp.maximum(acc + b1_ref[...], 0.0)
    h1_ref[...] = acc.reshape(bb, 64, 64, 128).astype(_BF)

    _im2col_s2(h1_ref[...], win2_ref, bb, 32)
    acc = jnp.dot(win2_ref[...].reshape(bb * 1024, 2048), w2_ref[...],
                  preferred_element_type=jnp.float32)
    h2 = jnp.maximum(acc + b2_ref[...], 0.0).astype(_BF).reshape(bb, 32, 32, 128)

    _im2col_s2(h2, win3_ref, bb, 16)
    acc = jnp.dot(win3_ref[...].reshape(bb * 256, 2048), w3_ref[...],
                  preferred_element_type=jnp.float32)
    h = acc + b3_ref[...]

    for w1_ref, w2_ref, relu_out in ((r0w1_ref, r0w2_ref, False),
                                     (r1w1_ref, r1w2_ref, True)):
        hr = jnp.maximum(h, 0.0).astype(_BF).reshape(bb, 16, 16, 128)
        _im2col_3x3(hr, rwin_ref, bb)
        t = jnp.dot(rwin_ref[...].reshape(bb * 256, 1152), w1_ref[...],
                    preferred_element_type=jnp.float32)
        t = jnp.maximum(t, 0.0).astype(_BF)
        h = h + jnp.dot(t, w2_ref[...], preferred_element_type=jnp.float32)
        if relu_out:
            h = jnp.maximum(h, 0.0)

    o_ref[...] = jnp.transpose(h.reshape(bb, 256, 128), (0, 2, 1))


def kernel(x_nchw, c1_w, c1_b, c2_w, c2_b, c3_w, c3_b,
           res0_w1, res0_w2, res1_w1, res1_w2):
    B = x_nchw.shape[0]
    h = jnp.transpose(x_nchw, (0, 2, 3, 1))
    hp = jnp.pad(h, ((0, 0), (1, 1), (1, 1), (0, 0)))
    hp = hp.reshape(B, 65, 2, 65, 2, 3)
    xs1 = jnp.transpose(hp, (0, 1, 3, 2, 4, 5)).reshape(B, 65, 65, 12)
    xs1 = xs1.astype(_BF)
    cols = [xs1[:, a:a + 64, b:b + 64, :] for a in range(2) for b in range(2)]
    p1 = jnp.concatenate(cols, axis=-1).reshape(B, 4096, 48)

    w1 = c1_w.astype(_BF)
    w2 = c2_w.astype(_BF)
    w3 = c3_w.astype(_BF)
    r0w1 = res0_w1.astype(_BF)
    r0w2 = res0_w2.astype(_BF)
    r1w1 = res1_w1.astype(_BF)
    r1w2 = res1_w2.astype(_BF)

    full = lambda shp: pl.BlockSpec(shp, lambda i: (0,) * len(shp))

    out = pl.pallas_call(
        functools.partial(_mega_body, bb=_BB),
        grid=(B // _BB,),
        in_specs=[
            pl.BlockSpec((_BB, 4096, 48), lambda i: (i, 0, 0)),
            full((48, 128)), full((1, 128)),
            full((2048, 128)), full((1, 128)),
            full((2048, 128)), full((1, 128)),
            full((1152, 128)), full((128, 128)),
            full((1152, 128)), full((128, 128)),
        ],
        out_shape=jax.ShapeDtypeStruct((B, 128, 256), jnp.float32),
        out_specs=pl.BlockSpec((_BB, 128, 256), lambda i: (i, 0, 0)),
        scratch_shapes=[
            pltpu.VMEM((_BB, 64, 64, 128), _BF),
            pltpu.VMEM((_BB, 32, 32, 2048), _BF),
            pltpu.VMEM((_BB, 16, 16, 2048), _BF),
            pltpu.VMEM((_BB, 16, 16, 1152), _BF),
        ],
        compiler_params=pltpu.CompilerParams(
            dimension_semantics=("arbitrary",)),
    )(p1, w1, c1_b, w2, c2_b, w3, c3_b, r0w1, r0w2, r1w1, r1w2)

    return out.reshape(B, 128, 16, 16)


# speedup vs baseline: 2.6085x; 1.0156x over previous
import functools

import jax
import jax.numpy as jnp
from jax.experimental import pallas as pl
from jax.experimental.pallas import tpu as pltpu

_BF = jnp.bfloat16
_BB = 4


def _s2d(x):
    B, H, W, C = x.shape
    xp = jnp.pad(x, ((0, 0), (1, 1), (1, 1), (0, 0)))
    Hi, Wi = (H + 2) // 2, (W + 2) // 2
    xp = xp.reshape(B, Hi, 2, Wi, 2, C)
    xp = jnp.transpose(xp, (0, 1, 3, 2, 4, 5))
    return xp.reshape(B, Hi, Wi, 4 * C)


def _repack(h, dst_ref, bb, hw):
    hv = h.reshape(bb, hw, 2, hw, 256)
    for dh in (0, 1):
        for dw in (0, 1):
            c0 = 128 * (2 * dh + dw)
            q = 1 - dw
            csl = hv[:, :, 1 - dh, :, q * 128:(q + 1) * 128]
            u0, v0 = 1 - dh, 1 - dw
            ur = (hw, hw + 1) if dh else (0, 1)
            vr = (hw, hw + 1) if dw else (0, 1)
            dst_ref[:, ur[0]:ur[1], :, c0:c0 + 128] = jnp.zeros(
                (bb, 1, hw + 1, 128), _BF)
            dst_ref[:, :, vr[0]:vr[1], c0:c0 + 128] = jnp.zeros(
                (bb, hw + 1, 1, 128), _BF)
            dst_ref[:, u0:u0 + hw, v0:v0 + hw, c0:c0 + 128] = csl


def _sconv(x, w_ref, bias, bb, hw):
    acc = bias
    for a in range(2):
        for b in range(2):
            sl = x[:, a:a + hw, b:b + hw, :].reshape(bb * hw * hw, 512)
            acc = acc + jnp.dot(sl, w_ref[2 * a + b],
                                preferred_element_type=jnp.float32)
    return acc


def _mega_body(p1_ref, w1_ref, b1_ref, w2_ref, b2_ref, w3_ref, b3_ref,
               r0w1_ref, r0w2_ref, r1w1_ref, r1w2_ref, o_ref,
               h1_ref, xs2_ref, xs3_ref, pad_ref, *, bb):
    acc = jnp.dot(p1_ref[...].reshape(bb * 4096, 48), w1_ref[...],
                  preferred_element_type=jnp.float32)
    acc = jnp.maximum(acc + b1_ref[...], 0.0)
    h1_ref[...] = acc.reshape(bb, 64, 64, 128).astype(_BF)

    _repack(h1_ref[...], xs2_ref, bb, 32)
    acc = _sconv(xs2_ref[...], w2_ref, b2_ref[...], bb, 32)
    h2 = jnp.maximum(acc, 0.0).astype(_BF).reshape(bb, 32, 32, 128)

    _repack(h2, xs3_ref, bb, 16)
    h = _sconv(xs3_ref[...], w3_ref, b3_ref[...], bb, 16)

    for w1_ref, w2_ref, relu_out in ((r0w1_ref, r0w2_ref, False),
                                     (r1w1_ref, r1w2_ref, True)):
        hr = jnp.maximum(h, 0.0).astype(_BF)
        pad_ref[...] = jnp.zeros(pad_ref.shape, _BF)
        pad_ref[:, 1:17, 1:17, :] = hr.reshape(bb, 16, 16, 128)
        xp = pad_ref[...]
        t = None
        for kh in range(3):
            for kw in range(3):
                sl = xp[:, kh:kh + 16, kw:kw + 16, :].reshape(bb * 256, 128)
                d = jnp.dot(sl, w1_ref[3 * kh + kw],
                            preferred_element_type=jnp.float32)
                t = d if t is None else t + d
        t = jnp.maximum(t, 0.0).astype(_BF)
        h = h + jnp.dot(t, w2_ref[...], preferred_element_type=jnp.float32)
        if relu_out:
            h = jnp.maximum(h, 0.0)

    o_ref[...] = jnp.transpose(h.reshape(bb, 256, 128), (0, 2, 1))


def kernel(x_nchw, c1_w, c1_b, c2_w, c2_b, c3_w, c3_b,
           res0_w1, res0_w2, res1_w1, res1_w2):
    B = x_nchw.shape[0]
    h = jnp.transpose(x_nchw.astype(_BF), (0, 2, 3, 1))
    xs1 = _s2d(h)
    cols = [xs1[:, a:a + 64, b:b + 64, :] for a in range(2) for b in range(2)]
    p1 = jnp.concatenate(cols, axis=-1).reshape(B, 4096, 48)

    w1 = c1_w.astype(_BF)
    w2 = c2_w.astype(_BF).reshape(4, 512, 128)
    w3 = c3_w.astype(_BF).reshape(4, 512, 128)
    r0w1 = res0_w1.astype(_BF).reshape(9, 128, 128)
    r0w2 = res0_w2.astype(_BF)
    r1w1 = res1_w1.astype(_BF).reshape(9, 128, 128)
    r1w2 = res1_w2.astype(_BF)

    full = lambda shp: pl.BlockSpec(shp, lambda i: (0,) * len(shp))

    out = pl.pallas_call(
        functools.partial(_mega_body, bb=_BB),
        grid=(B // _BB,),
        in_specs=[
            pl.BlockSpec((_BB, 4096, 48), lambda i: (i, 0, 0)),
            full((48, 128)), full((1, 128)),
            full((4, 512, 128)), full((1, 128)),
            full((4, 512, 128)), full((1, 128)),
            full((9, 128, 128)), full((128, 128)),
            full((9, 128, 128)), full((128, 128)),
        ],
        out_shape=jax.ShapeDtypeStruct((B, 128, 256), jnp.float32),
        out_specs=pl.BlockSpec((_BB, 128, 256), lambda i: (i, 0, 0)),
        scratch_shapes=[
            pltpu.VMEM((_BB, 64, 64, 128), _BF),
            pltpu.VMEM((_BB, 33, 33, 512), _BF),
            pltpu.VMEM((_BB, 17, 17, 512), _BF),
            pltpu.VMEM((_BB, 18, 18, 128), _BF),
        ],
        compiler_params=pltpu.CompilerParams(
            dimension_semantics=("parallel",)),
    )(p1, w1, c1_b, w2, c2_b, w3, c3_b, r0w1, r0w2, r1w1, r1w2)

    return out.reshape(B, 128, 16, 16)
